# baseline (speedup 1.0000x reference)
import functools

import jax
import jax.numpy as jnp
from jax import lax
from jax.experimental import pallas as pl
from jax.experimental.pallas import tpu as pltpu
from jax.experimental.pallas import tpu_sc as plsc

_CHUNK = 128
_NBUF = 4


@functools.lru_cache(maxsize=None)
def _build(fields, d, n_blocks, nc, ns):
    nw = nc * ns
    n_w = n_blocks // nw
    bt_tiles = n_blocks // fields
    dt = d // 8
    mesh = plsc.VectorSubcoreMesh(
        core_axis_name="c", subcore_axis_name="s",
        num_cores=nc, num_subcores=ns)

    @functools.partial(
        pl.kernel,
        out_type=jax.ShapeDtypeStruct(
            (fields, dt, bt_tiles, 8, _CHUNK), jnp.float32),
        mesh=mesh,
        compiler_params=pltpu.CompilerParams(
            use_tc_tiling_on_sc=False, needs_layout_passes=False),
        scratch_types=[
            pltpu.VMEM((n_w, _CHUNK), jnp.int32),
            pltpu.VMEM((_NBUF, _CHUNK, d), jnp.float32),
            pltpu.VMEM((_NBUF, d, _CHUNK), jnp.float32),
            pltpu.SemaphoreType.DMA,
            pltpu.SemaphoreType.DMA,
        ],
    )
    def gather_kernel(idx_hbm, table_hbm, out_hbm, idx_v, rows_v, t_v,
                      gsem, osem):
        wid = lax.axis_index("s") * nc + lax.axis_index("c")
        blk_base = wid * n_w
        pltpu.sync_copy(idx_hbm.at[pl.ds(blk_base, n_w)], idx_v)

        lanes = lax.iota(jnp.int32, 16)

        def issue(j, b):
            pltpu.async_copy(
                table_hbm.at[idx_v.at[j]], rows_v.at[b], gsem)

        def drain_gather(b):
            pltpu.make_async_copy(
                table_hbm.at[pl.ds(0, _CHUNK)], rows_v.at[b], gsem).wait()

        def transpose(b):
            rows = rows_v.at[b]
            dst = t_v.at[b]

            @pl.loop(0, d, unroll=4)
            def _(di):
                dvec = jnp.full((16,), di, jnp.int32)
                for bl0 in range(0, _CHUNK, 16):
                    v = plsc.load_gather(rows, [lanes + bl0, dvec])
                    dst[di, pl.ds(bl0, 16)] = v

        def store(j, b):
            blk = blk_base + j
            f = blk // bt_tiles
            bt = blk - f * bt_tiles
            for k in range(dt):
                pltpu.async_copy(
                    t_v.at[b].at[pl.ds(k * 8, 8)],
                    out_hbm.at[f, k, bt], osem)

        def drain_store(b):
            for k in range(dt):
                pltpu.make_async_copy(
                    t_v.at[b].at[pl.ds(k * 8, 8)],
                    out_hbm.at[0, k, 0], osem).wait()

        for b in range(_NBUF):
            issue(b, b)
        for b in range(_NBUF):
            drain_gather(b)
            transpose(b)
            issue(b + _NBUF, b)
            store(b, b)

        @pl.loop(_NBUF, n_w - _NBUF, step=_NBUF)
        def _(g):
            for b in range(_NBUF):
                j = g + b
                drain_gather(b)
                drain_store(b)
                transpose(b)
                issue(j + _NBUF, b)
                store(j, b)

        for b in range(_NBUF):
            j = n_w - _NBUF + b
            drain_gather(b)
            drain_store(b)
            transpose(b)
            store(j, b)
        for b in range(_NBUF):
            drain_store(b)

    return gather_kernel


def kernel(x, table):
    batch, fields = x.shape
    vocab, d = table.shape
    b_total = batch * fields
    idx = (x.T.astype(jnp.int32) * 2).reshape(b_total // _CHUNK, _CHUNK)
    tablep = jnp.concatenate(
        [table, jnp.zeros((vocab, 128 - d), jnp.float32)], axis=1)
    table2 = tablep.reshape(2 * vocab, d)
    fn = _build(fields, d, b_total // _CHUNK, 2, 16)
    out5 = fn(idx, table2)
    return out5.transpose(2, 4, 0, 1, 3).reshape(batch, fields, d)


# speedup vs baseline: 1.1149x; 1.1149x over previous
import functools

import jax
import jax.numpy as jnp
from jax import lax
from jax.experimental import pallas as pl
from jax.experimental.pallas import tpu as pltpu
from jax.experimental.pallas import tpu_sc as plsc

_CHUNK = 128
_NBUF = 4


@functools.lru_cache(maxsize=None)
def _build(fields, d, n_blocks, nc, ns):
    nw = nc * ns
    n_w = n_blocks // nw
    bt_tiles = n_blocks // fields
    dt = d // 8
    mesh = plsc.VectorSubcoreMesh(
        core_axis_name="c", subcore_axis_name="s",
        num_cores=nc, num_subcores=ns)

    @functools.partial(
        pl.kernel,
        out_type=jax.ShapeDtypeStruct(
            (fields, dt, bt_tiles, 8 * _CHUNK), jnp.float32),
        mesh=mesh,
        compiler_params=pltpu.CompilerParams(
            use_tc_tiling_on_sc=False, needs_layout_passes=False),
        scratch_types=[
            pltpu.VMEM((n_w, _CHUNK), jnp.int32),
            pltpu.VMEM((_NBUF, _CHUNK, d), jnp.float32),
            pltpu.VMEM((_NBUF, d * _CHUNK), jnp.float32),
            pltpu.SemaphoreType.DMA,
            pltpu.SemaphoreType.DMA,
        ],
    )
    def gather_kernel(idx_hbm, table_hbm, out_hbm, idx_v, rows_v, t_v,
                      gsem, osem):
        wid = lax.axis_index("s") * nc + lax.axis_index("c")
        blk_base = wid * n_w
        pltpu.sync_copy(idx_hbm.at[pl.ds(blk_base, n_w)], idx_v)

        lanes = lax.iota(jnp.int32, 16)

        def issue(j, b):
            pltpu.async_copy(
                table_hbm.at[idx_v.at[j]], rows_v.at[b], gsem)

        def drain_gather(b):
            pltpu.make_async_copy(
                table_hbm.at[pl.ds(0, _CHUNK)], rows_v.at[b], gsem).wait()

        def transpose(b):
            rows = rows_v.at[b]
            dst = t_v.at[b]
            cols = [(lanes + c * 16) * _CHUNK for c in range(d // 16)]

            @pl.loop(0, _CHUNK, unroll=4)
            def _(bl):
                for c in range(d // 16):
                    v = rows[bl, pl.ds(c * 16, 16)]
                    plsc.store_scatter(dst, [cols[c] + bl], v)

        def store(j, b):
            blk = blk_base + j
            f = blk // bt_tiles
            bt = blk - f * bt_tiles
            for k in range(dt):
                pltpu.async_copy(
                    t_v.at[b].at[pl.ds(k * 8 * _CHUNK, 8 * _CHUNK)],
                    out_hbm.at[f, k, bt], osem)

        def drain_store(b):
            for k in range(dt):
                pltpu.make_async_copy(
                    t_v.at[b].at[pl.ds(k * 8 * _CHUNK, 8 * _CHUNK)],
                    out_hbm.at[0, k, 0], osem).wait()

        for b in range(_NBUF):
            issue(b, b)
        for b in range(_NBUF):
            drain_gather(b)
            transpose(b)
            issue(b + _NBUF, b)
            store(b, b)

        @pl.loop(_NBUF, n_w - _NBUF, step=_NBUF)
        def _(g):
            for b in range(_NBUF):
                j = g + b
                drain_gather(b)
                drain_store(b)
                transpose(b)
                issue(j + _NBUF, b)
                store(j, b)

        for b in range(_NBUF):
            j = n_w - _NBUF + b
            drain_gather(b)
            drain_store(b)
            transpose(b)
            store(j, b)
        for b in range(_NBUF):
            drain_store(b)

    return gather_kernel


def kernel(x, table):
    batch, fields = x.shape
    vocab, d = table.shape
    b_total = batch * fields
    idx = (x.T.astype(jnp.int32) * 2).reshape(b_total // _CHUNK, _CHUNK)
    tablep = jnp.concatenate(
        [table, jnp.zeros((vocab, 128 - d), jnp.float32)], axis=1)
    table2 = tablep.reshape(2 * vocab, d)
    fn = _build(fields, d, b_total // _CHUNK, 2, 16)
    out5 = fn(idx, table2).reshape(fields, d // 8, b_total // _CHUNK // fields, 8, _CHUNK)
    return out5.transpose(2, 4, 0, 1, 3).reshape(batch, fields, d)


# speedup vs baseline: 1.5048x; 1.3498x over previous
import functools

import jax
import jax.numpy as jnp
from jax import lax
from jax.experimental import pallas as pl
from jax.experimental.pallas import tpu as pltpu
from jax.experimental.pallas import tpu_sc as plsc

_BROWS = 4
_NBUF = 4


@functools.lru_cache(maxsize=None)
def _build(batch, fields, d, nc, ns):
    nw = nc * ns
    chunk = _BROWS * fields
    n_chunks = batch // _BROWS
    n_w = n_chunks // nw
    mesh = plsc.VectorSubcoreMesh(
        core_axis_name="c", subcore_axis_name="s",
        num_cores=nc, num_subcores=ns)

    @functools.partial(
        pl.kernel,
        out_type=jax.ShapeDtypeStruct((batch, 32, 128), jnp.float32),
        mesh=mesh,
        compiler_params=pltpu.CompilerParams(
            use_tc_tiling_on_sc=False, needs_layout_passes=False),
        scratch_types=[
            pltpu.VMEM((n_w, chunk), jnp.int32),
            pltpu.VMEM((_NBUF, chunk, 128), jnp.float32),
            pltpu.SemaphoreType.DMA,
            pltpu.SemaphoreType.DMA,
        ],
    )
    def gather_kernel(idx_hbm, table_hbm, out_hbm, idx_v, rows_v, gsem, osem):
        wid = lax.axis_index("s") * nc + lax.axis_index("c")
        chunk_base = wid * n_w
        pltpu.sync_copy(idx_hbm.at[pl.ds(chunk_base, n_w)], idx_v)

        def issue(j, b):
            pltpu.async_copy(
                table_hbm.at[idx_v.at[j]], rows_v.at[b], gsem)

        def drain_gather(b):
            pltpu.make_async_copy(
                table_hbm.at[pl.ds(0, chunk)], rows_v.at[b], gsem).wait()

        def store(j, b):
            b0 = (chunk_base + j) * _BROWS
            for i in range(_BROWS):
                pltpu.async_copy(
                    rows_v.at[b].at[pl.ds(i * fields, fields)],
                    out_hbm.at[b0 + i, pl.ds(0, fields)], osem)

        def drain_store(b):
            for i in range(_BROWS):
                pltpu.make_async_copy(
                    rows_v.at[b].at[pl.ds(i * fields, fields)],
                    out_hbm.at[0, pl.ds(0, fields)], osem).wait()

        for b in range(_NBUF):
            issue(b, b)

        @pl.loop(0, n_w - _NBUF, step=_NBUF)
        def _(g):
            for b in range(_NBUF):
                j = g + b
                drain_gather(b)
                store(j, b)
                drain_store(b)
                issue(j + _NBUF, b)

        for b in range(_NBUF):
            j = n_w - _NBUF + b
            drain_gather(b)
            store(j, b)
            drain_store(b)

    return gather_kernel


def kernel(x, table):
    batch, fields = x.shape
    vocab, d = table.shape
    idx = x.astype(jnp.int32).reshape(batch // _BROWS, _BROWS * fields)
    tablep = jnp.concatenate(
        [table, jnp.zeros((vocab, 128 - d), jnp.float32)], axis=1)
    fn = _build(batch, fields, d, 2, 16)
    out = fn(idx, tablep)
    return out[:, :fields, :d]


# speedup vs baseline: 1.6605x; 1.1034x over previous
import functools

import jax
import jax.numpy as jnp
from jax import lax
from jax.experimental import pallas as pl
from jax.experimental.pallas import tpu as pltpu
from jax.experimental.pallas import tpu_sc as plsc

_BROWS = 4
_NBUF = 4


@functools.lru_cache(maxsize=None)
def _build(batch, fields, d, nc, ns):
    nw = nc * ns
    chunk = _BROWS * fields
    n_chunks = batch // _BROWS
    n_w = n_chunks // nw
    mesh = plsc.VectorSubcoreMesh(
        core_axis_name="c", subcore_axis_name="s",
        num_cores=nc, num_subcores=ns)

    @functools.partial(
        pl.kernel,
        out_type=jax.ShapeDtypeStruct((batch, 32, 128), jnp.float32),
        mesh=mesh,
        compiler_params=pltpu.CompilerParams(
            use_tc_tiling_on_sc=False, needs_layout_passes=False),
        scratch_types=[
            pltpu.VMEM((n_w, chunk), jnp.int32),
            pltpu.VMEM((_NBUF, chunk, d), jnp.float32),
            pltpu.SemaphoreType.DMA,
            pltpu.SemaphoreType.DMA,
        ],
    )
    def gather_kernel(idx_hbm, table_hbm, out_hbm, idx_v, rows_v, gsem, osem):
        wid = lax.axis_index("s") * nc + lax.axis_index("c")
        chunk_base = wid * n_w
        pltpu.sync_copy(idx_hbm.at[pl.ds(chunk_base, n_w)], idx_v)

        def issue(j, b):
            pltpu.async_copy(
                table_hbm.at[idx_v.at[j]], rows_v.at[b], gsem)

        def drain_gather(b):
            pltpu.make_async_copy(
                table_hbm.at[pl.ds(0, chunk)], rows_v.at[b], gsem).wait()

        def store(j, b):
            b0 = (chunk_base + j) * _BROWS
            for i in range(_BROWS):
                pltpu.async_copy(
                    rows_v.at[b].at[pl.ds(i * fields, fields)],
                    out_hbm.at[b0 + i, pl.ds(0, fields), pl.ds(0, d)], osem)

        def drain_store(b):
            for i in range(_BROWS):
                pltpu.make_async_copy(
                    rows_v.at[b].at[pl.ds(i * fields, fields)],
                    out_hbm.at[0, pl.ds(0, fields), pl.ds(0, d)], osem).wait()

        for b in range(_NBUF):
            issue(b, b)

        @pl.loop(0, n_w - _NBUF, step=_NBUF)
        def _(g):
            for b in range(_NBUF):
                j = g + b
                drain_gather(b)
                store(j, b)
                drain_store(b)
                issue(j + _NBUF, b)

        for b in range(_NBUF):
            j = n_w - _NBUF + b
            drain_gather(b)
            store(j, b)
            drain_store(b)

    return gather_kernel


def kernel(x, table):
    batch, fields = x.shape
    vocab, d = table.shape
    idx = (x.astype(jnp.int32) * 2).reshape(batch // _BROWS, _BROWS * fields)
    tablep = jnp.concatenate(
        [table, jnp.zeros((vocab, 128 - d), jnp.float32)], axis=1)
    table2 = tablep.reshape(2 * vocab, d)
    fn = _build(batch, fields, d, 2, 16)
    out = fn(idx, table2)
    return out[:, :fields, :d]
